# baseline (speedup 1.0000x reference)
import functools

import jax
import jax.numpy as jnp
from jax import lax
from jax.experimental import pallas as pl
from jax.experimental.pallas import tpu as pltpu
from jax.experimental.pallas import tpu_sc as plsc

_LANES = 16


def _sc_gather_v(idx_flat, table):
    B = idx_flat.shape[0]
    dim = table.shape[1]
    info = plsc.get_sparse_core_info()
    nw = info.num_cores * info.num_subcores
    b_per_w = B // nw
    assert b_per_w * nw == B and (b_per_w % 8) == 0

    mesh = plsc.VectorSubcoreMesh(core_axis_name="c", subcore_axis_name="s")

    @functools.partial(
        pl.kernel,
        mesh=mesh,
        compiler_params=pltpu.CompilerParams(use_tc_tiling_on_sc=True),
        out_type=[
            jax.ShapeDtypeStruct((B, dim), jnp.float32),
        ],
        scratch_types=[
            pltpu.VMEM((b_per_w,), jnp.int32),
            pltpu.VMEM((b_per_w, dim), jnp.float32),
            pltpu.SemaphoreType.DMA,
        ],
    )
    def body(idx_hbm, t_hbm, out_hbm, idx_v, rows, sem):
        wid = lax.axis_index("s") * info.num_cores + lax.axis_index("c")
        base = wid * b_per_w
        pltpu.sync_copy(idx_hbm.at[pl.ds(base, b_per_w)], idx_v)
        pltpu.async_copy(t_hbm.at[idx_v], rows, sem).wait()
        pltpu.sync_copy(rows, out_hbm.at[pl.ds(base, b_per_w)])

    (out,) = body(idx_flat, table)
    return out


def _sc_gather_k_pairs(idx_flat, pair_table, k_dim):
    B = idx_flat.shape[0]
    info = plsc.get_sparse_core_info()
    nw = info.num_cores * info.num_subcores
    b_per_w = B // nw
    assert b_per_w * nw == B and (b_per_w % 8) == 0

    mesh = plsc.VectorSubcoreMesh(core_axis_name="c", subcore_axis_name="s")

    chunk = 64
    n_chunks = b_per_w // chunk

    @functools.partial(
        pl.kernel,
        mesh=mesh,
        compiler_params=pltpu.CompilerParams(
            use_tc_tiling_on_sc=True, needs_layout_passes=False
        ),
        out_type=[
            jax.ShapeDtypeStruct((B, k_dim), jnp.float32),
        ],
        scratch_types=[
            pltpu.VMEM((b_per_w,), jnp.int32),
            pltpu.VMEM((chunk,), jnp.int32),
            pltpu.VMEM((chunk, 2 * k_dim), jnp.float32),
            pltpu.VMEM((chunk, k_dim), jnp.float32),
            pltpu.SemaphoreType.DMA,
        ],
    )
    def body(idx_hbm, t_hbm, out_hbm, idx_v, pidx_v, wide, rows, sem):
        wid = lax.axis_index("s") * info.num_cores + lax.axis_index("c")
        base = wid * b_per_w
        pltpu.sync_copy(idx_hbm.at[pl.ds(base, b_per_w)], idx_v)
        lane = lax.iota(jnp.int32, _LANES)

        def do_chunk(g, _):
            g0 = g * chunk

            def mk_pidx(i, _):
                v = idx_v[pl.ds(g0 + i * _LANES, _LANES)]
                pidx_v[pl.ds(i * _LANES, _LANES)] = lax.shift_right_logical(v, 1)
                return ()

            lax.fori_loop(0, chunk // _LANES, mk_pidx, ())
            pltpu.async_copy(t_hbm.at[pidx_v], wide, sem).wait()

            def sel(i, _):
                r0 = i * _LANES
                pars = lax.bitwise_and(idx_v[pl.ds(g0 + r0, _LANES)], 1)
                coff = pars * k_dim
                rr = r0 + lane
                for c in range(k_dim):
                    vals = plsc.load_gather(wide, [rr, coff + c])
                    plsc.store_scatter(
                        rows, [rr, jnp.full((_LANES,), c, jnp.int32)], vals
                    )
                return ()

            lax.fori_loop(0, chunk // _LANES, sel, ())
            pltpu.sync_copy(rows, out_hbm.at[pl.ds(base + g0, chunk)])
            return ()

        lax.fori_loop(0, n_chunks, do_chunk, ())

    (out,) = body(idx_flat, pair_table)
    return out


def kernel(idx, k_emb, v_emb):
    idx_flat = idx.reshape(-1).astype(jnp.int32)
    v_out = _sc_gather_v(idx_flat, v_emb)
    k_pairs = k_emb.reshape(k_emb.shape[0] // 2, 2 * k_emb.shape[1])
    k_out = _sc_gather_k_pairs(idx_flat, k_pairs, k_emb.shape[1])
    return (
        k_out.reshape(*idx.shape, k_emb.shape[1]),
        v_out.reshape(*idx.shape, v_emb.shape[1]),
    )


# speedup vs baseline: 1.3994x; 1.3994x over previous
import functools

import jax
import jax.numpy as jnp
from jax import lax
from jax.experimental import pallas as pl
from jax.experimental.pallas import tpu as pltpu
from jax.experimental.pallas import tpu_sc as plsc


def _sc_gather_v(idx_flat, table):
    B = idx_flat.shape[0]
    dim = table.shape[1]
    info = plsc.get_sparse_core_info()
    nw = info.num_cores * info.num_subcores
    b_per_w = B // nw
    assert b_per_w * nw == B and (b_per_w % 8) == 0

    mesh = plsc.VectorSubcoreMesh(core_axis_name="c", subcore_axis_name="s")

    @functools.partial(
        pl.kernel,
        mesh=mesh,
        compiler_params=pltpu.CompilerParams(use_tc_tiling_on_sc=True),
        out_type=[
            jax.ShapeDtypeStruct((B, dim), jnp.float32),
        ],
        scratch_types=[
            pltpu.VMEM((b_per_w,), jnp.int32),
            pltpu.VMEM((b_per_w, dim), jnp.float32),
            pltpu.SemaphoreType.DMA,
        ],
    )
    def body(idx_hbm, t_hbm, out_hbm, idx_v, rows, sem):
        wid = lax.axis_index("s") * info.num_cores + lax.axis_index("c")
        base = wid * b_per_w
        pltpu.sync_copy(idx_hbm.at[pl.ds(base, b_per_w)], idx_v)
        pltpu.async_copy(t_hbm.at[idx_v], rows, sem).wait()
        pltpu.sync_copy(rows, out_hbm.at[pl.ds(base, b_per_w)])

    (out,) = body(idx_flat, table)
    return out


def _sc_gather_k(idx_flat, table, n_batch):
    B = idx_flat.shape[0]
    dim = table.shape[1]
    info = plsc.get_sparse_core_info()
    nw = info.num_cores * info.num_subcores
    b_per_w = B // nw
    b_per_batch = B // n_batch
    w_per_batch = b_per_batch // b_per_w
    assert b_per_w * nw == B and (b_per_w % 8) == 0

    mesh = plsc.VectorSubcoreMesh(core_axis_name="c", subcore_axis_name="s")

    @functools.partial(
        pl.kernel,
        mesh=mesh,
        compiler_params=pltpu.CompilerParams(use_tc_tiling_on_sc=False),
        out_type=[
            jax.ShapeDtypeStruct((n_batch, b_per_batch, dim), jnp.float32),
        ],
        scratch_types=[
            pltpu.VMEM((b_per_w,), jnp.int32),
            pltpu.VMEM((b_per_w, dim), jnp.float32),
            pltpu.SemaphoreType.DMA,
        ],
    )
    def body(idx_hbm, t_hbm, out_hbm, idx_v, rows, sem):
        wid = lax.axis_index("s") * info.num_cores + lax.axis_index("c")
        base = wid * b_per_w
        pltpu.sync_copy(idx_hbm.at[pl.ds(base, b_per_w)], idx_v)
        pltpu.async_copy(t_hbm.at[idx_v], rows, sem).wait()
        bi = wid // w_per_batch
        j0 = (wid % w_per_batch) * b_per_w
        pltpu.sync_copy(rows, out_hbm.at[bi, pl.ds(j0, b_per_w)])

    (out,) = body(idx_flat, table)
    return out


def kernel(idx, k_emb, v_emb):
    idx_flat = idx.reshape(-1).astype(jnp.int32)
    v_out = _sc_gather_v(idx_flat, v_emb)
    k_out = _sc_gather_k(idx_flat, k_emb, idx.shape[0])
    return (
        k_out,
        v_out.reshape(*idx.shape, v_emb.shape[1]),
    )


# speedup vs baseline: 1.4162x; 1.0120x over previous
import functools

import jax
import jax.numpy as jnp
from jax import lax
from jax.experimental import pallas as pl
from jax.experimental.pallas import tpu as pltpu
from jax.experimental.pallas import tpu_sc as plsc


def _sc_gather_v(idx_flat, table):
    B = idx_flat.shape[0]
    dim = table.shape[1]
    info = plsc.get_sparse_core_info()
    nw = info.num_cores * info.num_subcores
    b_per_w = B // nw
    assert b_per_w * nw == B and (b_per_w % 8) == 0

    mesh = plsc.VectorSubcoreMesh(core_axis_name="c", subcore_axis_name="s")

    @functools.partial(
        pl.kernel,
        mesh=mesh,
        compiler_params=pltpu.CompilerParams(use_tc_tiling_on_sc=True),
        out_type=[
            jax.ShapeDtypeStruct((B, dim), jnp.float32),
        ],
        scratch_types=[
            pltpu.VMEM((b_per_w,), jnp.int32),
            pltpu.VMEM((b_per_w, dim), jnp.float32),
            pltpu.SemaphoreType.DMA,
        ],
    )
    def body(idx_hbm, t_hbm, out_hbm, idx_v, rows, sem):
        wid = lax.axis_index("s") * info.num_cores + lax.axis_index("c")
        base = wid * b_per_w
        pltpu.sync_copy(idx_hbm.at[pl.ds(base, b_per_w)], idx_v)
        pltpu.async_copy(t_hbm.at[idx_v], rows, sem).wait()
        pltpu.sync_copy(rows, out_hbm.at[pl.ds(base, b_per_w)])

    (out,) = body(idx_flat, table)
    return out


def _sc_gather_k(idx_flat, table, n_batch):
    B = idx_flat.shape[0]
    dim = table.shape[1]
    info = plsc.get_sparse_core_info()
    nw = info.num_cores * info.num_subcores
    b_per_w = B // nw
    b_per_batch = B // n_batch
    w_per_batch = b_per_batch // b_per_w
    assert b_per_w * nw == B and (b_per_w % 8) == 0

    mesh = plsc.VectorSubcoreMesh(core_axis_name="c", subcore_axis_name="s")

    @functools.partial(
        pl.kernel,
        mesh=mesh,
        compiler_params=pltpu.CompilerParams(use_tc_tiling_on_sc=False),
        out_type=[
            jax.ShapeDtypeStruct((n_batch, b_per_batch, dim), jnp.float32),
        ],
        scratch_types=[
            pltpu.VMEM((b_per_w,), jnp.int32),
            pltpu.VMEM((b_per_w, dim), jnp.float32),
            pltpu.SemaphoreType.DMA,
        ],
    )
    def body(idx_hbm, t_hbm, out_hbm, idx_v, rows, sem):
        wid = lax.axis_index("s") * info.num_cores + lax.axis_index("c")
        base = wid * b_per_w
        pltpu.sync_copy(idx_hbm.at[pl.ds(base, b_per_w)], idx_v)
        pltpu.async_copy(t_hbm.at[idx_v], rows, sem).wait()
        bi = wid // w_per_batch
        j0 = (wid % w_per_batch) * b_per_w
        pltpu.sync_copy(rows, out_hbm.at[bi, pl.ds(j0, b_per_w)])

    (out,) = body(idx_flat, table)
    return out


def kernel(idx, k_emb, v_emb):
    idx_flat = idx.reshape(-1).astype(jnp.int32)
    v_out = _sc_gather_v(idx_flat, v_emb)
    idx_flat2, _ = lax.optimization_barrier((idx_flat, v_out))
    k_out = _sc_gather_k(idx_flat2, k_emb, idx.shape[0])
    return (
        k_out,
        v_out.reshape(*idx.shape, v_emb.shape[1]),
    )
